# Initial kernel scaffold; baseline (speedup 1.0000x reference)
import jax
import jax.numpy as jnp
from jax.experimental import pallas as pl


def kernel(topo, weight, params):
    raise NotImplementedError("write your pallas kernel here")


# speedup vs baseline: 117.9198x; 117.9198x over previous
import jax
import jax.numpy as jnp
from jax.experimental import pallas as pl

_B, _N, _H, _NH, _HD, _DEPTH, _EHD, _MLPH = 1, 256, 256, 8, 32, 2, 64, 1024
_SCALE = _HD ** -0.5
_EDGE_ROWS = 64


def _ln(x):
    m = jnp.mean(x, axis=-1, keepdims=True)
    d = x - m
    v = jnp.mean(d * d, axis=-1, keepdims=True)
    return d / jnp.sqrt(v + 1e-6)


def _gelu(x):
    c = 0.7978845608028654
    return 0.5 * x * (1.0 + jnp.tanh(c * (x + 0.044715 * (x * x * x))))


def _dott(a, b):
    return jax.lax.dot_general(a, b, (((1,), (1,)), ((), ())),
                               preferred_element_type=jnp.float32)


def _node_body(*refs):
    (topo_ref, topot_ref, w_ref, tw_ref, tb_ref,
     wtw_ref, wtb_ref, ne_ref) = refs[:8]
    out_ref = refs[-1]
    blk = refs[8:-1]

    mt = topot_ref[:] != 0.0
    x = _dott(topo_ref[:], tw_ref[:]) + tb_ref[:]
    x = x + w_ref[:] * wtw_ref[:] + wtb_ref[:]
    x = x + ne_ref[:]

    for b in range(_DEPTH):
        (qw, qb, kw, kb, vw, vb, ow, ob,
         m1w, m1b, m2w, m2b) = [r[:] for r in blk[b * 12:(b + 1) * 12]]
        h = _ln(x)
        q = _dott(h, qw) + qb
        k = _dott(h, kw) + kb
        v = _dott(h, vw) + vb
        heads = []
        for t in range(_NH):
            s = slice(t * _HD, (t + 1) * _HD)
            sc = _dott(q[:, s], k[:, s]) * _SCALE
            sc = jnp.where(mt, sc, -jnp.inf)
            mx = jnp.max(sc, axis=1, keepdims=True)
            ex = jnp.where(mt, jnp.exp(sc - mx), 0.0)
            sm = jnp.sum(ex, axis=1, keepdims=True)
            at = ex / (sm + 1e-12)
            heads.append(jnp.dot(at, v[:, s],
                                 preferred_element_type=jnp.float32))
        o = jnp.concatenate(heads, axis=1)
        x = _dott(o, ow) + ob + x
        h2 = _ln(x)
        mm = _gelu(_dott(h2, m1w) + m1b)
        x = _dott(mm, m2w) + m2b + x
    out_ref[:] = x


def _edge_body(xb_ref, x_ref, w1a_ref, w1b_ref, w1c_ref, e1b_ref,
               e2w_ref, e2b_ref, e3w_ref, e3b_ref, out_ref):
    r = xb_ref.shape[0]
    xb = xb_ref[:]
    xf = x_ref[:]
    a = _dott(xb, w1a_ref[:]) + e1b_ref[:]
    bm = _dott(xf, w1b_ref[:])
    p = (xb[:, None, :] * xf[None, :, :]).reshape(r * _N, _H)
    c = _dott(p, w1c_ref[:]).reshape(r, _N, _EHD)
    l1 = _gelu(c + a[:, None, :] + bm[None, :, :])
    l2 = _gelu(_dott(l1.reshape(r * _N, _EHD), e2w_ref[:]) + e2b_ref[:])
    lg = jnp.sum(l2 * e3w_ref[:], axis=1, keepdims=True) + e3b_ref[:]
    out_ref[:] = jax.nn.sigmoid(lg).reshape(r, _N)


def kernel(topo, weight, params):
    p = params
    t2 = topo.reshape(_N, _N)
    row = lambda arr: arr.reshape(1, -1)

    ops = [t2, t2.T, weight.reshape(_N, 1), p['topo_w'], row(p['topo_b']),
           p['wt_w'].reshape(1, _H), row(p['wt_b']), p['n_emb']]
    for blk in p['blocks']:
        ops += [blk['qw'], row(blk['qb']), blk['kw'], row(blk['kb']),
                blk['vw'], row(blk['vb']), blk['ow'], row(blk['ob']),
                blk['m1w'], row(blk['m1b']), blk['m2w'], row(blk['m2b'])]

    x = pl.pallas_call(
        _node_body,
        out_shape=jax.ShapeDtypeStruct((_N, _H), jnp.float32),
    )(*ops)

    e1w = p['e1w']
    w1a, w1b, w1c = e1w[:, :_H], e1w[:, _H:2 * _H], e1w[:, 2 * _H:]
    full = lambda shp: pl.BlockSpec(shp, lambda i: (0, 0))
    r = _EDGE_ROWS
    out = pl.pallas_call(
        _edge_body,
        grid=(_N // r,),
        in_specs=[
            pl.BlockSpec((r, _H), lambda i: (i, 0)),
            full((_N, _H)),
            full((_EHD, _H)), full((_EHD, _H)), full((_EHD, _H)),
            full((1, _EHD)),
            full((_EHD, _EHD)), full((1, _EHD)),
            full((1, _EHD)), full((1, 1)),
        ],
        out_specs=pl.BlockSpec((r, _N), lambda i: (i, 0)),
        out_shape=jax.ShapeDtypeStruct((_N, _N), jnp.float32),
    )(x, x, w1a, w1b, w1c, row(p['e1b']), p['e2w'], row(p['e2b']),
      p['e3w'], p['e3b'].reshape(1, 1))
    return out.reshape(_B, _N, _N)


# speedup vs baseline: 312.5763x; 2.6508x over previous
import jax
import jax.numpy as jnp
from jax.experimental import pallas as pl

_B, _N, _H, _NH, _HD, _DEPTH, _EHD, _MLPH = 1, 256, 256, 8, 32, 2, 64, 1024
_SCALE = _HD ** -0.5
_R = 64
_NCHUNK = (_N // 2) // _R


def _ln(x):
    m = jnp.mean(x, axis=-1, keepdims=True)
    d = x - m
    v = jnp.mean(d * d, axis=-1, keepdims=True)
    return d / jnp.sqrt(v + 1e-6)


def _gelu(x):
    c = 0.7978845608028654
    return 0.5 * x * (1.0 + jnp.tanh(c * (x + 0.044715 * (x * x * x))))


def _dott(a, b):
    return jax.lax.dot_general(a, b, (((1,), (1,)), ((), ())),
                               preferred_element_type=jnp.float32)


def _dot(a, b):
    return jnp.dot(a, b, preferred_element_type=jnp.float32)


def _fused_body(*refs):
    (topo_ref, topot_ref, w_ref, tw_ref, tb_ref,
     wtw_ref, wtb_ref, ne_ref) = refs[:8]
    blk = refs[8:8 + 12 * _DEPTH]
    (w1a_ref, w1b_ref, wlo_ref, whi_ref, e1b_ref,
     w2p_ref, e2bp_ref, e3p_ref, e3b_ref) = refs[8 + 12 * _DEPTH:-1]
    out_ref = refs[-1]

    mt = topot_ref[:] != 0.0
    x = _dott(topo_ref[:], tw_ref[:]) + tb_ref[:]
    x = x + w_ref[:] * wtw_ref[:] + wtb_ref[:]
    x = x + ne_ref[:]

    for b in range(_DEPTH):
        (qw, qb, kw, kb, vw, vb, ow, ob,
         m1w, m1b, m2w, m2b) = [r[:] for r in blk[b * 12:(b + 1) * 12]]
        h = _ln(x)
        q = _dott(h, qw) + qb
        k = _dott(h, kw) + kb
        v = _dott(h, vw) + vb
        heads = []
        for t in range(_NH):
            s = slice(t * _HD, (t + 1) * _HD)
            sc = _dott(q[:, s], k[:, s]) * _SCALE
            sc = jnp.where(mt, sc, -jnp.inf)
            mx = jnp.max(sc, axis=1, keepdims=True)
            ex = jnp.where(mt, jnp.exp(sc - mx), 0.0)
            sm = jnp.sum(ex, axis=1, keepdims=True)
            at = ex / (sm + 1e-12)
            heads.append(_dot(at, v[:, s]))
        o = jnp.concatenate(heads, axis=1)
        x = _dott(o, ow) + ob + x
        h2 = _ln(x)
        mm = _gelu(_dott(h2, m1w) + m1b)
        x = _dott(mm, m2w) + m2b + x

    a_full = _dott(x, w1a_ref[:]) + e1b_ref[:]
    bm = _dott(x, w1b_ref[:])
    b_pk = jnp.concatenate([bm, bm], axis=1).astype(jnp.bfloat16)
    x16 = x.astype(jnp.bfloat16)
    e2bp = e2bp_ref[:].astype(jnp.bfloat16)
    e3b = e3b_ref[0, 0]
    half = _N // 2
    for g in range(_NCHUNK):
        lo = slice(g * _R, (g + 1) * _R)
        hi = slice(half + g * _R, half + (g + 1) * _R)
        a_pk = jnp.concatenate([a_full[lo], a_full[hi]],
                               axis=1).astype(jnp.bfloat16)
        p_lo = (x16[lo][:, None, :] * x16[None, :, :]).reshape(_R * _N, _H)
        p_hi = (x16[hi][:, None, :] * x16[None, :, :]).reshape(_R * _N, _H)
        c = _dot(p_lo, wlo_ref[:]) + _dot(p_hi, whi_ref[:])
        l1 = (c.astype(jnp.bfloat16).reshape(_R, _N, 2 * _EHD)
              + a_pk[:, None, :] + b_pk[None, :, :])
        l1b = _gelu(l1).reshape(_R * _N, 2 * _EHD)
        l2b = _gelu(_dot(l1b, w2p_ref[:]).astype(jnp.bfloat16) + e2bp)
        lgp = _dot(l2b, e3p_ref[:])
        sg = jax.nn.sigmoid(lgp[:, 0:2] + e3b)
        out_ref[0, g * _R:(g + 1) * _R, :] = sg[:, 0:1].reshape(_R, _N)
        out_ref[1, g * _R:(g + 1) * _R, :] = sg[:, 1:2].reshape(_R, _N)


def kernel(topo, weight, params):
    p = params
    t2 = topo.reshape(_N, _N)
    row = lambda arr: arr.reshape(1, -1)

    ops = [t2, t2.T, weight.reshape(_N, 1), p['topo_w'], row(p['topo_b']),
           p['wt_w'].reshape(1, _H), row(p['wt_b']), p['n_emb']]
    for blk in p['blocks']:
        ops += [blk['qw'], row(blk['qb']), blk['kw'], row(blk['kb']),
                blk['vw'], row(blk['vb']), blk['ow'], row(blk['ob']),
                blk['m1w'], row(blk['m1b']), blk['m2w'], row(blk['m2b'])]

    e1w = p['e1w']
    w1a, w1b, w1c = e1w[:, :_H], e1w[:, _H:2 * _H], e1w[:, 2 * _H:]
    zpad = jnp.zeros((_H, _EHD), jnp.bfloat16)
    w1ct = w1c.T.astype(jnp.bfloat16)
    wlo = jnp.concatenate([w1ct, zpad], axis=1)
    whi = jnp.concatenate([zpad, w1ct], axis=1)
    e2t = p['e2w'].T.astype(jnp.bfloat16)
    z64 = jnp.zeros((_EHD, _EHD), jnp.bfloat16)
    w2p = jnp.concatenate([jnp.concatenate([e2t, z64], axis=1),
                           jnp.concatenate([z64, e2t], axis=1)], axis=0)
    e2bp = jnp.concatenate([row(p['e2b']), row(p['e2b'])], axis=1)
    e3col = p['e3w'].reshape(_EHD, 1).astype(jnp.bfloat16)
    z1 = jnp.zeros((_EHD, 1), jnp.bfloat16)
    zrest = jnp.zeros((2 * _EHD, 2 * _EHD - 2), jnp.bfloat16)
    e3p = jnp.concatenate([
        jnp.concatenate([e3col, z1], axis=1),
        jnp.concatenate([z1, e3col], axis=1)], axis=0)
    e3p = jnp.concatenate([e3p, zrest], axis=1)

    ops += [w1a, w1b, wlo, whi, row(p['e1b']), w2p, e2bp, e3p,
            p['e3b'].reshape(1, 1)]

    out = pl.pallas_call(
        _fused_body,
        out_shape=jax.ShapeDtypeStruct((2, _N // 2, _N), jnp.float32),
    )(*ops)
    return out.reshape(_B, _N, _N)


# speedup vs baseline: 321.6863x; 1.0291x over previous
import jax
import jax.numpy as jnp
from jax.experimental import pallas as pl

_B, _N, _H, _NH, _HD, _DEPTH, _EHD, _MLPH = 1, 256, 256, 8, 32, 2, 64, 1024
_SCALE = _HD ** -0.5
_R = 64
_NCHUNK = (_N // 2) // _R


def _ln(x):
    m = jnp.mean(x, axis=-1, keepdims=True)
    d = x - m
    v = jnp.mean(d * d, axis=-1, keepdims=True)
    return d / jnp.sqrt(v + 1e-6)


def _gelu(x):
    c = 0.7978845608028654
    return 0.5 * x * (1.0 + jnp.tanh(c * (x + 0.044715 * (x * x * x))))


def _dott(a, b):
    return jax.lax.dot_general(a, b, (((1,), (1,)), ((), ())),
                               preferred_element_type=jnp.float32)


def _dot(a, b):
    return jnp.dot(a, b, preferred_element_type=jnp.float32)


def _dott16(a, b):
    return jax.lax.dot_general(a.astype(jnp.bfloat16), b.astype(jnp.bfloat16),
                               (((1,), (1,)), ((), ())),
                               preferred_element_type=jnp.float32)


def _fused_body(*refs):
    (topo_ref, topot_ref, w_ref, tw_ref, tb_ref,
     wtw_ref, wtb_ref, ne_ref) = refs[:8]
    blk = refs[8:8 + 12 * _DEPTH]
    (w1a_ref, w1b_ref, wlo_ref, whi_ref, e1b_ref,
     w2p_ref, e2bp_ref, e3p_ref, e3b_ref) = refs[8 + 12 * _DEPTH:-1]
    out_ref = refs[-1]

    mt = topot_ref[:] != 0.0
    x = _dott(topo_ref[:], tw_ref[:]) + tb_ref[:]
    x = x + w_ref[:] * wtw_ref[:] + wtb_ref[:]
    x = x + ne_ref[:]

    for b in range(_DEPTH):
        (qw, qb, kw, kb, vw, vb, ow, ob,
         m1w, m1b, m2w, m2b) = [r[:] for r in blk[b * 12:(b + 1) * 12]]
        h = _ln(x)
        q = _dott16(h, qw) + qb
        k = _dott16(h, kw) + kb
        v = (_dott16(h, vw) + vb).astype(jnp.bfloat16)
        heads = []
        for t in range(_NH):
            s = slice(t * _HD, (t + 1) * _HD)
            sc = _dott16(q[:, s], k[:, s]) * _SCALE
            sc = jnp.where(mt, sc, -jnp.inf)
            mx = jnp.max(sc, axis=1, keepdims=True)
            ex = jnp.where(mt, jnp.exp(sc - mx), 0.0)
            sm = jnp.sum(ex, axis=1, keepdims=True)
            at = ex / (sm + 1e-12)
            heads.append(jax.lax.dot_general(
                at.astype(jnp.bfloat16), v[:, s], (((1,), (0,)), ((), ())),
                preferred_element_type=jnp.float32))
        o = jnp.concatenate(heads, axis=1)
        x = _dott16(o, ow) + ob + x
        h2 = _ln(x)
        mm = _gelu(_dott16(h2, m1w) + m1b)
        x = _dott16(mm, m2w) + m2b + x

    a_full = _dott(x, w1a_ref[:]) + e1b_ref[:]
    bm = _dott(x, w1b_ref[:])
    b_pk = jnp.concatenate([bm, bm], axis=1).astype(jnp.bfloat16)
    x16 = x.astype(jnp.bfloat16)
    e2bp = e2bp_ref[:].astype(jnp.bfloat16)
    e3b = e3b_ref[0, 0]
    half = _N // 2
    for g in range(_NCHUNK):
        lo = slice(g * _R, (g + 1) * _R)
        hi = slice(half + g * _R, half + (g + 1) * _R)
        a_pk = jnp.concatenate([a_full[lo], a_full[hi]],
                               axis=1).astype(jnp.bfloat16)
        p_lo = (x16[lo][:, None, :] * x16[None, :, :]).reshape(_R * _N, _H)
        p_hi = (x16[hi][:, None, :] * x16[None, :, :]).reshape(_R * _N, _H)
        c = _dot(p_lo, wlo_ref[:]) + _dot(p_hi, whi_ref[:])
        l1 = (c.astype(jnp.bfloat16).reshape(_R, _N, 2 * _EHD)
              + a_pk[:, None, :] + b_pk[None, :, :])
        l1b = _gelu(l1).reshape(_R * _N, 2 * _EHD)
        l2b = _gelu(_dot(l1b, w2p_ref[:]).astype(jnp.bfloat16) + e2bp)
        lgp = _dot(l2b, e3p_ref[:])
        sg = jax.nn.sigmoid(lgp[:, 0:2] + e3b)
        out_ref[0, g * _R:(g + 1) * _R, :] = sg[:, 0:1].reshape(_R, _N)
        out_ref[1, g * _R:(g + 1) * _R, :] = sg[:, 1:2].reshape(_R, _N)


def kernel(topo, weight, params):
    p = params
    t2 = topo.reshape(_N, _N)
    row = lambda arr: arr.reshape(1, -1)

    ops = [t2, t2.T, weight.reshape(_N, 1), p['topo_w'], row(p['topo_b']),
           p['wt_w'].reshape(1, _H), row(p['wt_b']), p['n_emb']]
    for blk in p['blocks']:
        ops += [blk['qw'], row(blk['qb']), blk['kw'], row(blk['kb']),
                blk['vw'], row(blk['vb']), blk['ow'], row(blk['ob']),
                blk['m1w'], row(blk['m1b']), blk['m2w'], row(blk['m2b'])]

    e1w = p['e1w']
    w1a, w1b, w1c = e1w[:, :_H], e1w[:, _H:2 * _H], e1w[:, 2 * _H:]
    zpad = jnp.zeros((_H, _EHD), jnp.bfloat16)
    w1ct = w1c.T.astype(jnp.bfloat16)
    wlo = jnp.concatenate([w1ct, zpad], axis=1)
    whi = jnp.concatenate([zpad, w1ct], axis=1)
    e2t = p['e2w'].T.astype(jnp.bfloat16)
    z64 = jnp.zeros((_EHD, _EHD), jnp.bfloat16)
    w2p = jnp.concatenate([jnp.concatenate([e2t, z64], axis=1),
                           jnp.concatenate([z64, e2t], axis=1)], axis=0)
    e2bp = jnp.concatenate([row(p['e2b']), row(p['e2b'])], axis=1)
    e3col = p['e3w'].reshape(_EHD, 1).astype(jnp.bfloat16)
    z1 = jnp.zeros((_EHD, 1), jnp.bfloat16)
    zrest = jnp.zeros((2 * _EHD, 2 * _EHD - 2), jnp.bfloat16)
    e3p = jnp.concatenate([
        jnp.concatenate([e3col, z1], axis=1),
        jnp.concatenate([z1, e3col], axis=1)], axis=0)
    e3p = jnp.concatenate([e3p, zrest], axis=1)

    ops += [w1a, w1b, wlo, whi, row(p['e1b']), w2p, e2bp, e3p,
            p['e3b'].reshape(1, 1)]

    out = pl.pallas_call(
        _fused_body,
        out_shape=jax.ShapeDtypeStruct((2, _N // 2, _N), jnp.float32),
    )(*ops)
    return out.reshape(_B, _N, _N)


# speedup vs baseline: 324.8721x; 1.0099x over previous
import jax
import jax.numpy as jnp
from jax.experimental import pallas as pl

_B, _N, _H, _NH, _HD, _DEPTH, _EHD, _MLPH = 1, 256, 256, 8, 32, 2, 64, 1024
_SCALE = _HD ** -0.5
_R = 64
_NCHUNK = (_N // 2) // _R


def _ln(x):
    m = jnp.mean(x, axis=-1, keepdims=True)
    d = x - m
    v = jnp.mean(d * d, axis=-1, keepdims=True)
    return d / jnp.sqrt(v + 1e-6)


def _gelu(x):
    c = 0.7978845608028654
    return 0.5 * x * (1.0 + jnp.tanh(c * (x + 0.044715 * (x * x * x))))


def _dott(a, b):
    return jax.lax.dot_general(a, b, (((1,), (1,)), ((), ())),
                               preferred_element_type=jnp.float32)


def _dot(a, b):
    return jnp.dot(a, b, preferred_element_type=jnp.float32)


def _dott16(a, b):
    return jax.lax.dot_general(a.astype(jnp.bfloat16), b.astype(jnp.bfloat16),
                               (((1,), (1,)), ((), ())),
                               preferred_element_type=jnp.float32)


def _fused_body(*refs):
    (topo_ref, topot_ref, w_ref, tw_ref, tb_ref,
     wtw_ref, wtb_ref, ne_ref) = refs[:8]
    blk = refs[8:8 + 12 * _DEPTH]
    (w1a_ref, w1b_ref, w1ct_ref, e1b_ref,
     w2p_ref, e2bp_ref, e3p_ref, e3b_ref) = refs[8 + 12 * _DEPTH:-1]
    out_ref = refs[-1]

    mt = topot_ref[:] != 0.0
    x = _dott(topo_ref[:], tw_ref[:]) + tb_ref[:]
    x = x + w_ref[:] * wtw_ref[:] + wtb_ref[:]
    x = x + ne_ref[:]

    for b in range(_DEPTH):
        (qw, qb, kw, kb, vw, vb, ow, ob,
         m1w, m1b, m2w, m2b) = [r[:] for r in blk[b * 12:(b + 1) * 12]]
        h = _ln(x)
        q = _dott16(h, qw) + qb
        k = _dott16(h, kw) + kb
        v = (_dott16(h, vw) + vb).astype(jnp.bfloat16)
        heads = []
        for t in range(_NH):
            s = slice(t * _HD, (t + 1) * _HD)
            sc = _dott16(q[:, s], k[:, s]) * _SCALE
            sc = jnp.where(mt, sc, -jnp.inf)
            mx = jnp.max(sc, axis=1, keepdims=True)
            ex = jnp.where(mt, jnp.exp(sc - mx), 0.0)
            sm = jnp.sum(ex, axis=1, keepdims=True)
            at = ex / (sm + 1e-12)
            heads.append(jax.lax.dot_general(
                at.astype(jnp.bfloat16), v[:, s], (((1,), (0,)), ((), ())),
                preferred_element_type=jnp.float32))
        o = jnp.concatenate(heads, axis=1)
        x = _dott16(o, ow) + ob + x
        h2 = _ln(x)
        mm = _gelu(_dott16(h2, m1w) + m1b)
        x = _dott16(mm, m2w) + m2b + x

    a_full = _dott(x, w1a_ref[:]) + e1b_ref[:]
    bm = _dott(x, w1b_ref[:])
    b_pk = jnp.concatenate([bm, bm], axis=1).astype(jnp.bfloat16)
    x16 = x.astype(jnp.bfloat16)
    xt16 = x.T.astype(jnp.bfloat16)
    w1ct = w1ct_ref[:]
    e2bp = e2bp_ref[:].astype(jnp.bfloat16)
    e3b = e3b_ref[0, 0]
    half = _N // 2
    for g in range(_NCHUNK):
        lo0 = g * _R
        hi0 = half + g * _R
        a_pk = jnp.concatenate([a_full[lo0:lo0 + _R], a_full[hi0:hi0 + _R]],
                               axis=1).astype(jnp.bfloat16)
        parts = []
        for r in range(_R):
            parts.append(w1ct * xt16[:, lo0 + r:lo0 + r + 1])
            parts.append(w1ct * xt16[:, hi0 + r:hi0 + r + 1])
        w_big = jnp.concatenate(parts, axis=1)
        c3 = _dot(x16, w_big).reshape(_N, _R, 2 * _EHD)
        l1 = c3.astype(jnp.bfloat16) + a_pk[None, :, :] + b_pk[:, None, :]
        l1b = _gelu(l1).reshape(_N * _R, 2 * _EHD)
        l2b = _gelu(_dot(l1b, w2p_ref[:]).astype(jnp.bfloat16) + e2bp)
        lgp = _dot(l2b, e3p_ref[:])
        sg = jax.nn.sigmoid(lgp[:, 0:2] + e3b)
        out_ref[0, lo0:lo0 + _R, :] = sg[:, 0:1].reshape(_N, _R).T
        out_ref[1, lo0:lo0 + _R, :] = sg[:, 1:2].reshape(_N, _R).T


def kernel(topo, weight, params):
    p = params
    t2 = topo.reshape(_N, _N)
    row = lambda arr: arr.reshape(1, -1)

    ops = [t2, t2.T, weight.reshape(_N, 1), p['topo_w'], row(p['topo_b']),
           p['wt_w'].reshape(1, _H), row(p['wt_b']), p['n_emb']]
    for blk in p['blocks']:
        ops += [blk['qw'], row(blk['qb']), blk['kw'], row(blk['kb']),
                blk['vw'], row(blk['vb']), blk['ow'], row(blk['ob']),
                blk['m1w'], row(blk['m1b']), blk['m2w'], row(blk['m2b'])]

    e1w = p['e1w']
    w1a, w1b, w1c = e1w[:, :_H], e1w[:, _H:2 * _H], e1w[:, 2 * _H:]
    w1ct = w1c.T.astype(jnp.bfloat16)
    e2t = p['e2w'].T.astype(jnp.bfloat16)
    z64 = jnp.zeros((_EHD, _EHD), jnp.bfloat16)
    w2p = jnp.concatenate([jnp.concatenate([e2t, z64], axis=1),
                           jnp.concatenate([z64, e2t], axis=1)], axis=0)
    e2bp = jnp.concatenate([row(p['e2b']), row(p['e2b'])], axis=1)
    e3col = p['e3w'].reshape(_EHD, 1).astype(jnp.bfloat16)
    z1 = jnp.zeros((_EHD, 1), jnp.bfloat16)
    zrest = jnp.zeros((2 * _EHD, 2 * _EHD - 2), jnp.bfloat16)
    e3p = jnp.concatenate([
        jnp.concatenate([e3col, z1], axis=1),
        jnp.concatenate([z1, e3col], axis=1)], axis=0)
    e3p = jnp.concatenate([e3p, zrest], axis=1)

    ops += [w1a, w1b, w1ct, row(p['e1b']), w2p, e2bp, e3p,
            p['e3b'].reshape(1, 1)]

    out = pl.pallas_call(
        _fused_body,
        out_shape=jax.ShapeDtypeStruct((2, _N // 2, _N), jnp.float32),
    )(*ops)
    return out.reshape(_B, _N, _N)


# speedup vs baseline: 377.8447x; 1.1631x over previous
import jax
import jax.numpy as jnp
from jax.experimental import pallas as pl

_B, _N, _H, _NH, _HD, _DEPTH, _EHD, _MLPH = 1, 256, 256, 8, 32, 2, 64, 1024
_SCALE = _HD ** -0.5
_R = 64
_NCHUNK = (_N // 2) // _R


def _ln(x):
    m = jnp.mean(x, axis=-1, keepdims=True)
    d = x - m
    v = jnp.mean(d * d, axis=-1, keepdims=True)
    return d / jnp.sqrt(v + 1e-6)


def _gelu(x):
    c = 0.7978845608028654
    return 0.5 * x * (1.0 + jnp.tanh(c * (x + 0.044715 * (x * x * x))))


def _dott(a, b):
    return jax.lax.dot_general(a, b, (((1,), (1,)), ((), ())),
                               preferred_element_type=jnp.float32)


def _dot(a, b):
    return jnp.dot(a, b, preferred_element_type=jnp.float32)


def _dott16(a, b):
    return jax.lax.dot_general(a.astype(jnp.bfloat16), b.astype(jnp.bfloat16),
                               (((1,), (1,)), ((), ())),
                               preferred_element_type=jnp.float32)


def _fused_body(*refs):
    (topo_ref, w_ref, tw_ref, tb_ref,
     wtw_ref, wtb_ref, ne_ref) = refs[:7]
    blk = refs[7:7 + 12 * _DEPTH]
    (e1w_ref, e1b_ref, e2w_ref, e2b_ref, e3w_ref, e3b_ref) \
        = refs[7 + 12 * _DEPTH:-1]
    out_ref = refs[-1]

    mt = topo_ref[:].T != 0.0
    x = _dott16(topo_ref[:], tw_ref[:]) + tb_ref[:]
    x = x + w_ref[:] * wtw_ref[:] + wtb_ref[:]
    x = x + ne_ref[:]

    for b in range(_DEPTH):
        (qw, qb, kw, kb, vw, vb, ow, ob,
         m1w, m1b, m2w, m2b) = [r[:] for r in blk[b * 12:(b + 1) * 12]]
        h = _ln(x)
        q = _dott16(h, qw) + qb
        k = _dott16(h, kw) + kb
        v = (_dott16(h, vw) + vb).astype(jnp.bfloat16)
        heads = []
        for t in range(_NH):
            s = slice(t * _HD, (t + 1) * _HD)
            sc = _dott16(q[:, s], k[:, s]) * _SCALE
            sc = jnp.where(mt, sc, -jnp.inf)
            mx = jnp.max(sc, axis=1, keepdims=True)
            ex = jnp.where(mt, jnp.exp(sc - mx), 0.0)
            sm = jnp.sum(ex, axis=1, keepdims=True)
            at = ex / (sm + 1e-12)
            heads.append(jax.lax.dot_general(
                at.astype(jnp.bfloat16), v[:, s], (((1,), (0,)), ((), ())),
                preferred_element_type=jnp.float32))
        o = jnp.concatenate(heads, axis=1)
        x = _dott16(o, ow) + ob + x
        h2 = _ln(x)
        mm = _gelu(_dott16(h2, m1w) + m1b)
        x = _dott16(mm, m2w) + m2b + x

    e1w = e1w_ref[:]
    bm = _dott16(x, e1w[:, _H:2 * _H])
    b_pk = jnp.concatenate([bm, bm], axis=1).astype(jnp.bfloat16)
    x16 = x.astype(jnp.bfloat16)
    xt16 = x.T.astype(jnp.bfloat16)
    w1ct = e1w[:, 2 * _H:].T.astype(jnp.bfloat16)
    e2t = e2w_ref[:].T.astype(jnp.bfloat16)
    z64 = jnp.zeros((_EHD, _EHD), jnp.bfloat16)
    w2p = jnp.concatenate([jnp.concatenate([e2t, z64], axis=1),
                           jnp.concatenate([z64, e2t], axis=1)], axis=0)
    e2bp = jnp.concatenate([e2b_ref[:], e2b_ref[:]],
                           axis=1).astype(jnp.bfloat16)
    e3col = e3w_ref[:].T.astype(jnp.bfloat16)
    z1 = jnp.zeros((_EHD, 1), jnp.bfloat16)
    e3p = jnp.concatenate([
        jnp.concatenate([e3col, z1], axis=1),
        jnp.concatenate([z1, e3col], axis=1)], axis=0)
    e3p = jnp.concatenate(
        [e3p, jnp.zeros((2 * _EHD, 2 * _EHD - 2), jnp.bfloat16)], axis=1)
    e3b = e3b_ref[0, 0]
    rc = 2 * _R
    hp = rc // 2
    w1ct_t = jnp.tile(w1ct, (1, rc))
    w1at_t = jnp.tile(e1w[:, :_H].T.astype(jnp.bfloat16), (1, rc))
    e1b_t = jnp.tile(e1b_ref[:], (1, rc))
    ones16 = jnp.ones((1, _H), jnp.bfloat16)
    for g in range(_NCHUNK):
        base = g * rc
        xe = jnp.repeat(xt16[:, base:base + rc], _EHD, axis=1)
        w_big = w1ct_t * xe
        c3 = _dot(x16, w_big).reshape(_N, hp, 2 * _EHD)
        a_flat = _dot(ones16, w1at_t * xe) + e1b_t
        a3 = a_flat.astype(jnp.bfloat16).reshape(1, hp, 2 * _EHD)
        l1 = c3.astype(jnp.bfloat16) + a3 + b_pk[:, None, :]
        l1b = _gelu(l1).reshape(_N * hp, 2 * _EHD)
        l2b = _gelu(_dot(l1b, w2p).astype(jnp.bfloat16) + e2bp)
        lgp = _dot(l2b, e3p)
        ev = lgp[:, 0:1].reshape(_N, hp).T
        od = lgp[:, 1:2].reshape(_N, hp).T
        rows = jnp.stack([ev, od], axis=1).reshape(rc, _N)
        out_ref[base:base + rc, :] = jax.nn.sigmoid(rows + e3b)


def kernel(topo, weight, params):
    p = params
    t2 = topo.reshape(_N, _N)
    row = lambda arr: arr.reshape(1, -1)

    ops = [t2, weight.reshape(_N, 1), p['topo_w'], row(p['topo_b']),
           p['wt_w'].reshape(1, _H), row(p['wt_b']), p['n_emb']]
    for blk in p['blocks']:
        ops += [blk['qw'], row(blk['qb']), blk['kw'], row(blk['kb']),
                blk['vw'], row(blk['vb']), blk['ow'], row(blk['ob']),
                blk['m1w'], row(blk['m1b']), blk['m2w'], row(blk['m2b'])]
    ops += [p['e1w'], row(p['e1b']), p['e2w'], row(p['e2b']),
            p['e3w'], p['e3b'].reshape(1, 1)]

    out = pl.pallas_call(
        _fused_body,
        out_shape=jax.ShapeDtypeStruct((_N, _N), jnp.float32),
    )(*ops)
    return out.reshape(_B, _N, _N)


# speedup vs baseline: 407.7901x; 1.0793x over previous
import jax
import jax.numpy as jnp
from jax.experimental import pallas as pl

_B, _N, _H, _NH, _HD, _DEPTH, _EHD, _MLPH = 1, 256, 256, 8, 32, 2, 64, 1024
_SCALE = _HD ** -0.5
_R = 64
_NCHUNK = (_N // 2) // _R


def _ln(x):
    m = jnp.mean(x, axis=-1, keepdims=True)
    d = x - m
    v = jnp.mean(d * d, axis=-1, keepdims=True)
    return d / jnp.sqrt(v + 1e-6)


def _gelu(x):
    c = 0.7978845608028654
    return 0.5 * x * (1.0 + jnp.tanh(c * (x + 0.044715 * (x * x * x))))


def _dott(a, b):
    return jax.lax.dot_general(a, b, (((1,), (1,)), ((), ())),
                               preferred_element_type=jnp.float32)


def _dot(a, b):
    return jnp.dot(a, b, preferred_element_type=jnp.float32)


def _dott16(a, b):
    return jax.lax.dot_general(a.astype(jnp.bfloat16), b.astype(jnp.bfloat16),
                               (((1,), (1,)), ((), ())),
                               preferred_element_type=jnp.float32)


def _fused_body(*refs):
    (topo_ref, w_ref, tw_ref, tb_ref,
     wtw_ref, wtb_ref, ne_ref) = refs[:7]
    blk = refs[7:7 + 12 * _DEPTH]
    (e1w_ref, e1b_ref, e2w_ref, e2b_ref, e3w_ref, e3b_ref) \
        = refs[7 + 12 * _DEPTH:-1]
    out_ref = refs[-1]

    mt = topo_ref[:].T != 0.0
    x = _dott16(topo_ref[:], tw_ref[:]) + tb_ref[:]
    x = x + w_ref[:] * wtw_ref[:] + wtb_ref[:]
    x = x + ne_ref[:]

    for b in range(_DEPTH):
        (qw, qb, kw, kb, vw, vb, ow, ob,
         m1w, m1b, m2w, m2b) = [r[:] for r in blk[b * 12:(b + 1) * 12]]
        h = _ln(x)
        q = _dott16(h, qw) + qb
        k = _dott16(h, kw) + kb
        v = (_dott16(h, vw) + vb).astype(jnp.bfloat16)
        heads = []
        for t in range(_NH):
            s = slice(t * _HD, (t + 1) * _HD)
            sc = _dott16(q[:, s], k[:, s]) * _SCALE
            sc = jnp.where(mt, sc, -jnp.inf)
            mx = jnp.max(sc, axis=1, keepdims=True)
            ex = jnp.where(mt, jnp.exp(sc - mx), 0.0)
            sm = jnp.sum(ex, axis=1, keepdims=True)
            at = ex / (sm + 1e-12)
            heads.append(jax.lax.dot_general(
                at.astype(jnp.bfloat16), v[:, s], (((1,), (0,)), ((), ())),
                preferred_element_type=jnp.float32))
        o = jnp.concatenate(heads, axis=1)
        x = _dott16(o, ow) + ob + x
        h2 = _ln(x)
        mm = _gelu(_dott16(h2, m1w) + m1b)
        x = _dott16(mm, m2w) + m2b + x

    e1w = e1w_ref[:]
    a_full = _dott16(x, e1w[:, :_H]) + e1b_ref[:]
    bm = _dott16(x, e1w[:, _H:2 * _H])
    b_pk = jnp.concatenate([bm, bm], axis=1).astype(jnp.bfloat16)
    x16 = x.astype(jnp.bfloat16)
    xt16 = x.T.astype(jnp.bfloat16)
    w1ct = e1w[:, 2 * _H:].T.astype(jnp.bfloat16)
    e2t = e2w_ref[:].T.astype(jnp.bfloat16)
    z64 = jnp.zeros((_EHD, _EHD), jnp.bfloat16)
    w2p = jnp.concatenate([jnp.concatenate([e2t, z64], axis=1),
                           jnp.concatenate([z64, e2t], axis=1)], axis=0)
    e2bp = jnp.concatenate([e2b_ref[:], e2b_ref[:]],
                           axis=1).astype(jnp.bfloat16)
    e3col = e3w_ref[:].T.astype(jnp.bfloat16)
    z1 = jnp.zeros((_EHD, 1), jnp.bfloat16)
    e3p = jnp.concatenate([
        jnp.concatenate([e3col, z1], axis=1),
        jnp.concatenate([z1, e3col], axis=1)], axis=0)
    e3p = jnp.concatenate(
        [e3p, jnp.zeros((2 * _EHD, 2 * _EHD - 2), jnp.bfloat16)], axis=1)
    e3b = e3b_ref[0, 0]
    half = _N // 2
    for g in range(_NCHUNK):
        lo0 = g * _R
        hi0 = half + g * _R
        a_pk = jnp.concatenate([a_full[lo0:lo0 + _R], a_full[hi0:hi0 + _R]],
                               axis=1).astype(jnp.bfloat16)
        parts = []
        for r in range(_R):
            parts.append(w1ct * xt16[:, lo0 + r:lo0 + r + 1])
            parts.append(w1ct * xt16[:, hi0 + r:hi0 + r + 1])
        w_big = jnp.concatenate(parts, axis=1)
        c3 = _dot(x16, w_big).reshape(_N, _R, 2 * _EHD)
        l1 = c3.astype(jnp.bfloat16) + a_pk[None, :, :] + b_pk[:, None, :]
        l1b = _gelu(l1).reshape(_N * _R, 2 * _EHD)
        l2b = _gelu(_dot(l1b, w2p).astype(jnp.bfloat16) + e2bp)
        lgp = _dot(l2b, e3p)
        lg_lo = lgp[:, 0:1].reshape(_N, _R).T
        lg_hi = lgp[:, 1:2].reshape(_N, _R).T
        out_ref[0, lo0:lo0 + _R, :] = jax.nn.sigmoid(lg_lo + e3b)
        out_ref[1, lo0:lo0 + _R, :] = jax.nn.sigmoid(lg_hi + e3b)


def kernel(topo, weight, params):
    p = params
    t2 = topo.reshape(_N, _N)
    row = lambda arr: arr.reshape(1, -1)

    ops = [t2, weight.reshape(_N, 1), p['topo_w'], row(p['topo_b']),
           p['wt_w'].reshape(1, _H), row(p['wt_b']), p['n_emb']]
    for blk in p['blocks']:
        ops += [blk['qw'], row(blk['qb']), blk['kw'], row(blk['kb']),
                blk['vw'], row(blk['vb']), blk['ow'], row(blk['ob']),
                blk['m1w'], row(blk['m1b']), blk['m2w'], row(blk['m2b'])]
    ops += [p['e1w'], row(p['e1b']), p['e2w'], row(p['e2b']),
            p['e3w'], p['e3b'].reshape(1, 1)]

    out = pl.pallas_call(
        _fused_body,
        out_shape=jax.ShapeDtypeStruct((2, _N // 2, _N), jnp.float32),
    )(*ops)
    return out.reshape(_B, _N, _N)
